# baseline (speedup 1.0000x reference)
import jax
import jax.numpy as jnp
from jax import lax
from jax.experimental import pallas as pl
from jax.experimental.pallas import tpu as pltpu

NN = 10000
KK = 16
DH = 256
NPAD = 10240

RBLK = 256
MB = 512


def _tau_body(pos_blk_ref, post_ref, tau_ref):
    i = pl.program_id(0)
    pb = pos_blk_ref[...]
    pt = post_ref[...]
    sqi = jnp.sum(pb * pb, axis=1, keepdims=True)
    sqj = jnp.sum(pt * pt, axis=0, keepdims=True)
    dot = jnp.dot(pb, pt, preferred_element_type=jnp.float32)
    d = sqi + sqj - 2.0 * dot
    rows = i * RBLK + lax.broadcasted_iota(jnp.int32, (RBLK, NPAD), 0)
    cols = lax.broadcasted_iota(jnp.int32, (RBLK, NPAD), 1)
    inf = jnp.float32(jnp.inf)
    d = jnp.where(cols == rows, inf, d)
    m = None
    for _ in range(KK):
        m = jnp.min(d, axis=1, keepdims=True)
        d = jnp.where(d == m, inf, d)
    tau_ref[...] = (m + (sqi + 1.0) * 4.8e-7) * 0.5


def _tau_call(pos_p, post):
    grid = NPAD // RBLK
    return pl.pallas_call(
        _tau_body,
        grid=(grid,),
        in_specs=[
            pl.BlockSpec((RBLK, 3), lambda i: (i, 0)),
            pl.BlockSpec((3, NPAD), lambda i: (0, 0)),
        ],
        out_specs=pl.BlockSpec((RBLK, 1), lambda i: (i, 0)),
        out_shape=jax.ShapeDtypeStruct((NPAD, 1), jnp.float32),
    )(pos_p, post)


AB = 2048


def _aggmm_body(pos_blk_ref, post_ref, tau_ref, taut_ref, v_ref,
                o_ref, cnt_ref):
    j = pl.program_id(1)
    pb = pos_blk_ref[...]
    pt = post_ref[...]
    sqhi = 0.5 * jnp.sum(pb * pb, axis=1, keepdims=True)
    sqhj = 0.5 * jnp.sum(pt * pt, axis=0, keepdims=True)
    dot = jnp.dot(pb, pt, preferred_element_type=jnp.float32)
    thrm = (sqhi + sqhj) - jnp.maximum(tau_ref[...], taut_ref[...])
    mf = (dot >= thrm).astype(jnp.float32)
    o = jnp.dot(mf, v_ref[...], preferred_element_type=jnp.float32)
    cnt = jnp.dot(mf, jnp.ones((AB, 1), jnp.float32),
                  preferred_element_type=jnp.float32)

    @pl.when(j == 0)
    def _():
        o_ref[...] = o
        cnt_ref[...] = cnt

    @pl.when(j > 0)
    def _():
        o_ref[...] += o
        cnt_ref[...] += cnt


def _aggmm_call(pos_p, post_p, tau, taut, v):
    grid = (NPAD // AB, NPAD // AB)
    return pl.pallas_call(
        _aggmm_body,
        grid=grid,
        in_specs=[
            pl.BlockSpec((AB, 3), lambda i, j: (i, 0)),
            pl.BlockSpec((3, AB), lambda i, j: (0, j)),
            pl.BlockSpec((AB, 1), lambda i, j: (i, 0)),
            pl.BlockSpec((1, AB), lambda i, j: (0, j)),
            pl.BlockSpec((AB, DH), lambda i, j: (j, 0)),
        ],
        out_specs=[
            pl.BlockSpec((AB, DH), lambda i, j: (i, 0)),
            pl.BlockSpec((AB, 1), lambda i, j: (i, 0)),
        ],
        out_shape=[
            jax.ShapeDtypeStruct((NPAD, DH), jnp.float32),
            jax.ShapeDtypeStruct((NPAD, 1), jnp.float32),
        ],
        compiler_params=pltpu.CompilerParams(
            dimension_semantics=("parallel", "arbitrary")),
    )(pos_p, post_p, tau, taut, v)


def _pre_body(x_ref, pos_ref, wa_ref, wb_ref, wc_ref, b_ref, u_ref, v_ref):
    xb = x_ref[...]
    posb = pos_ref[...]
    posc = (posb[:, 0:1] * wc_ref[0:1, :] + posb[:, 1:2] * wc_ref[1:2, :]
            + posb[:, 2:3] * wc_ref[2:3, :])
    u = jnp.dot(xb, wa_ref[...], preferred_element_type=jnp.float32)
    v = jnp.dot(xb, wb_ref[...], preferred_element_type=jnp.float32)
    u_ref[...] = u + b_ref[...] - posc
    v_ref[...] = v + posc


def _pre_call(x_p, pos_p, wa, wb, wc, b):
    grid = NPAD // MB
    return pl.pallas_call(
        _pre_body,
        grid=(grid,),
        in_specs=[
            pl.BlockSpec((MB, DH), lambda i: (i, 0)),
            pl.BlockSpec((MB, 3), lambda i: (i, 0)),
            pl.BlockSpec((DH, DH), lambda i: (0, 0)),
            pl.BlockSpec((DH, DH), lambda i: (0, 0)),
            pl.BlockSpec((3, DH), lambda i: (0, 0)),
            pl.BlockSpec((1, DH), lambda i: (0, 0)),
        ],
        out_specs=[
            pl.BlockSpec((MB, DH), lambda i: (i, 0)),
            pl.BlockSpec((MB, DH), lambda i: (i, 0)),
        ],
        out_shape=[
            jax.ShapeDtypeStruct((NPAD, DH), jnp.float32),
            jax.ShapeDtypeStruct((NPAD, DH), jnp.float32),
        ],
    )(x_p, pos_p, wa, wb, wc, b)


def _mid_body(u_ref, o_ref, v_ref, cnt_ref, pos_ref, wa_ref, wb_ref, wc_ref,
              b_ref, u2_ref, v2_ref):
    cnt = jnp.maximum(cnt_ref[...] - 1.0, 1.0)
    h = jnp.maximum(u_ref[...] + (o_ref[...] - v_ref[...]) / cnt, 0.0)
    posb = pos_ref[...]
    posc = (posb[:, 0:1] * wc_ref[0:1, :] + posb[:, 1:2] * wc_ref[1:2, :]
            + posb[:, 2:3] * wc_ref[2:3, :])
    u = jnp.dot(h, wa_ref[...], preferred_element_type=jnp.float32)
    v = jnp.dot(h, wb_ref[...], preferred_element_type=jnp.float32)
    u2_ref[...] = u + b_ref[...] - posc
    v2_ref[...] = v + posc


def _mid_call(u0, o0, v0, cnt, pos_p, wa, wb, wc, b):
    grid = NPAD // MB
    return pl.pallas_call(
        _mid_body,
        grid=(grid,),
        in_specs=[
            pl.BlockSpec((MB, DH), lambda i: (i, 0)),
            pl.BlockSpec((MB, DH), lambda i: (i, 0)),
            pl.BlockSpec((MB, DH), lambda i: (i, 0)),
            pl.BlockSpec((MB, 1), lambda i: (i, 0)),
            pl.BlockSpec((MB, 3), lambda i: (i, 0)),
            pl.BlockSpec((DH, DH), lambda i: (0, 0)),
            pl.BlockSpec((DH, DH), lambda i: (0, 0)),
            pl.BlockSpec((3, DH), lambda i: (0, 0)),
            pl.BlockSpec((1, DH), lambda i: (0, 0)),
        ],
        out_specs=[
            pl.BlockSpec((MB, DH), lambda i: (i, 0)),
            pl.BlockSpec((MB, DH), lambda i: (i, 0)),
        ],
        out_shape=[
            jax.ShapeDtypeStruct((NPAD, DH), jnp.float32),
            jax.ShapeDtypeStruct((NPAD, DH), jnp.float32),
        ],
    )(u0, o0, v0, cnt, pos_p, wa, wb, wc, b)


def _logits_body(u_ref, o_ref, v_ref, cnt_ref, wf_ref, bf_ref, out_ref):
    cnt = jnp.maximum(cnt_ref[...] - 1.0, 1.0)
    h = jnp.maximum(u_ref[...] + (o_ref[...] - v_ref[...]) / cnt, 0.0)
    out_ref[...] = (jnp.dot(h, wf_ref[...], preferred_element_type=jnp.float32)
                    + bf_ref[...])


def _logits_call(u1, o1, v1, cnt, wf, bf):
    grid = NPAD // MB
    return pl.pallas_call(
        _logits_body,
        grid=(grid,),
        in_specs=[
            pl.BlockSpec((MB, DH), lambda i: (i, 0)),
            pl.BlockSpec((MB, DH), lambda i: (i, 0)),
            pl.BlockSpec((MB, DH), lambda i: (i, 0)),
            pl.BlockSpec((MB, 1), lambda i: (i, 0)),
            pl.BlockSpec((DH, 1), lambda i: (0, 0)),
            pl.BlockSpec((1, 1), lambda i: (0, 0)),
        ],
        out_specs=pl.BlockSpec((MB, 1), lambda i: (i, 0)),
        out_shape=jax.ShapeDtypeStruct((NN, 1), jnp.float32),
    )(u1, o1, v1, cnt, wf, bf)


def _softmax_body(l_ref, p_ref):
    lg = l_ref[...]
    m = jnp.max(lg, axis=0, keepdims=True)
    e = jnp.exp(lg - m)
    p_ref[...] = e / jnp.sum(e, axis=0, keepdims=True)


def _softmax_call(logits):
    return pl.pallas_call(
        _softmax_body,
        out_shape=jax.ShapeDtypeStruct((NN, 1), jnp.float32),
    )(logits)


@jax.jit
def kernel(x, pos, W0, b0, W1, b1, Wf, bf):
    if pos.ndim == 3:
        pos = pos.mean(axis=1)
    x = x.astype(jnp.float32)
    pos = pos.astype(jnp.float32)

    x_p = jnp.pad(x, ((0, NPAD - NN), (0, 0)))
    pos_p = jnp.concatenate(
        [pos, jnp.full((NPAD - NN, 3), 1e9, jnp.float32)], axis=0)
    post = pos.T
    post_p = pos_p.T

    tau = _tau_call(pos_p, post_p)
    taut = jnp.where(jnp.arange(NPAD)[None, :] < NN,
                     tau.reshape(1, NPAD), jnp.float32(-3e38))

    wa0, wb0, wc0 = W0[:DH], W0[DH:2 * DH], W0[2 * DH:]
    u0, v0 = _pre_call(x_p, pos_p, wa0, wb0, wc0, b0.reshape(1, DH))
    o0, cnt = _aggmm_call(pos_p, post_p, tau, taut, v0)

    wa1, wb1, wc1 = W1[:DH], W1[DH:2 * DH], W1[2 * DH:]
    u1, v1 = _mid_call(u0, o0, v0, cnt, pos_p, wa1, wb1, wc1,
                       b1.reshape(1, DH))
    o1, _ = _aggmm_call(pos_p, post_p, tau, taut, v1)

    logits = _logits_call(u1, o1, v1, cnt, Wf, bf.reshape(1, 1))
    probs = _softmax_call(logits)
    return probs[:, 0]


# speedup vs baseline: 1.0039x; 1.0039x over previous
import jax
import jax.numpy as jnp
from jax import lax
from jax.experimental import pallas as pl
from jax.experimental.pallas import tpu as pltpu

NN = 10000
KK = 16
DH = 256
NPAD = 10240

RBLK = 256
MB = 512


def _tau_body(pos_blk_ref, post_ref, tau_ref):
    i = pl.program_id(0)
    pb = pos_blk_ref[...]
    pt = post_ref[...]
    sqi = jnp.sum(pb * pb, axis=1, keepdims=True)
    sqj = jnp.sum(pt * pt, axis=0, keepdims=True)
    dot = jnp.dot(pb, pt, preferred_element_type=jnp.float32)
    d = sqi + sqj - 2.0 * dot
    rows = i * RBLK + lax.broadcasted_iota(jnp.int32, (RBLK, NN), 0)
    cols = lax.broadcasted_iota(jnp.int32, (RBLK, NN), 1)
    inf = jnp.float32(jnp.inf)
    d = jnp.where(cols == rows, inf, d)
    m = None
    for _ in range(KK):
        m = jnp.min(d, axis=1, keepdims=True)
        d = jnp.where(d == m, inf, d)
    tau_ref[...] = (m + (sqi + 1.0) * 4.8e-7) * 0.5


def _tau_call(pos_p, post):
    grid = NPAD // RBLK
    return pl.pallas_call(
        _tau_body,
        grid=(grid,),
        in_specs=[
            pl.BlockSpec((RBLK, 3), lambda i: (i, 0)),
            pl.BlockSpec((3, NN), lambda i: (0, 0)),
        ],
        out_specs=pl.BlockSpec((RBLK, 1), lambda i: (i, 0)),
        out_shape=jax.ShapeDtypeStruct((NPAD, 1), jnp.float32),
    )(pos_p, post)


AB = 2048


def _aggmm_body(pos_blk_ref, post_ref, tau_ref, taut_ref, v_ref,
                o_ref, cnt_ref):
    j = pl.program_id(1)
    pb = pos_blk_ref[...]
    pt = post_ref[...]
    sqhi = 0.5 * jnp.sum(pb * pb, axis=1, keepdims=True)
    sqhj = 0.5 * jnp.sum(pt * pt, axis=0, keepdims=True)
    dot = jnp.dot(pb, pt, preferred_element_type=jnp.float32)
    thrm = (sqhi + sqhj) - jnp.maximum(tau_ref[...], taut_ref[...])
    mf = (dot >= thrm).astype(jnp.float32)
    o = jnp.dot(mf, v_ref[...], preferred_element_type=jnp.float32)
    cnt = jnp.dot(mf, jnp.ones((AB, 1), jnp.float32),
                  preferred_element_type=jnp.float32)

    @pl.when(j == 0)
    def _():
        o_ref[...] = o
        cnt_ref[...] = cnt

    @pl.when(j > 0)
    def _():
        o_ref[...] += o
        cnt_ref[...] += cnt


def _aggmm_call(pos_p, post_p, tau, taut, v):
    grid = (NPAD // AB, NPAD // AB)
    return pl.pallas_call(
        _aggmm_body,
        grid=grid,
        in_specs=[
            pl.BlockSpec((AB, 3), lambda i, j: (i, 0)),
            pl.BlockSpec((3, AB), lambda i, j: (0, j)),
            pl.BlockSpec((AB, 1), lambda i, j: (i, 0)),
            pl.BlockSpec((1, AB), lambda i, j: (0, j)),
            pl.BlockSpec((AB, DH), lambda i, j: (j, 0)),
        ],
        out_specs=[
            pl.BlockSpec((AB, DH), lambda i, j: (i, 0)),
            pl.BlockSpec((AB, 1), lambda i, j: (i, 0)),
        ],
        out_shape=[
            jax.ShapeDtypeStruct((NPAD, DH), jnp.float32),
            jax.ShapeDtypeStruct((NPAD, 1), jnp.float32),
        ],
        compiler_params=pltpu.CompilerParams(
            dimension_semantics=("parallel", "arbitrary")),
    )(pos_p, post_p, tau, taut, v)


def _pre_body(x_ref, pos_ref, wa_ref, wb_ref, wc_ref, b_ref, u_ref, v_ref):
    xb = x_ref[...]
    posb = pos_ref[...]
    posc = (posb[:, 0:1] * wc_ref[0:1, :] + posb[:, 1:2] * wc_ref[1:2, :]
            + posb[:, 2:3] * wc_ref[2:3, :])
    u = jnp.dot(xb, wa_ref[...], preferred_element_type=jnp.float32)
    v = jnp.dot(xb, wb_ref[...], preferred_element_type=jnp.float32)
    u_ref[...] = u + b_ref[...] - posc
    v_ref[...] = v + posc


def _pre_call(x_p, pos_p, wa, wb, wc, b):
    grid = NPAD // MB
    return pl.pallas_call(
        _pre_body,
        grid=(grid,),
        in_specs=[
            pl.BlockSpec((MB, DH), lambda i: (i, 0)),
            pl.BlockSpec((MB, 3), lambda i: (i, 0)),
            pl.BlockSpec((DH, DH), lambda i: (0, 0)),
            pl.BlockSpec((DH, DH), lambda i: (0, 0)),
            pl.BlockSpec((3, DH), lambda i: (0, 0)),
            pl.BlockSpec((1, DH), lambda i: (0, 0)),
        ],
        out_specs=[
            pl.BlockSpec((MB, DH), lambda i: (i, 0)),
            pl.BlockSpec((MB, DH), lambda i: (i, 0)),
        ],
        out_shape=[
            jax.ShapeDtypeStruct((NPAD, DH), jnp.float32),
            jax.ShapeDtypeStruct((NPAD, DH), jnp.float32),
        ],
    )(x_p, pos_p, wa, wb, wc, b)


def _mid_body(u_ref, o_ref, v_ref, cnt_ref, pos_ref, wa_ref, wb_ref, wc_ref,
              b_ref, u2_ref, v2_ref):
    cnt = jnp.maximum(cnt_ref[...] - 1.0, 1.0)
    h = jnp.maximum(u_ref[...] + (o_ref[...] - v_ref[...]) / cnt, 0.0)
    posb = pos_ref[...]
    posc = (posb[:, 0:1] * wc_ref[0:1, :] + posb[:, 1:2] * wc_ref[1:2, :]
            + posb[:, 2:3] * wc_ref[2:3, :])
    u = jnp.dot(h, wa_ref[...], preferred_element_type=jnp.float32)
    v = jnp.dot(h, wb_ref[...], preferred_element_type=jnp.float32)
    u2_ref[...] = u + b_ref[...] - posc
    v2_ref[...] = v + posc


def _mid_call(u0, o0, v0, cnt, pos_p, wa, wb, wc, b):
    grid = NPAD // MB
    return pl.pallas_call(
        _mid_body,
        grid=(grid,),
        in_specs=[
            pl.BlockSpec((MB, DH), lambda i: (i, 0)),
            pl.BlockSpec((MB, DH), lambda i: (i, 0)),
            pl.BlockSpec((MB, DH), lambda i: (i, 0)),
            pl.BlockSpec((MB, 1), lambda i: (i, 0)),
            pl.BlockSpec((MB, 3), lambda i: (i, 0)),
            pl.BlockSpec((DH, DH), lambda i: (0, 0)),
            pl.BlockSpec((DH, DH), lambda i: (0, 0)),
            pl.BlockSpec((3, DH), lambda i: (0, 0)),
            pl.BlockSpec((1, DH), lambda i: (0, 0)),
        ],
        out_specs=[
            pl.BlockSpec((MB, DH), lambda i: (i, 0)),
            pl.BlockSpec((MB, DH), lambda i: (i, 0)),
        ],
        out_shape=[
            jax.ShapeDtypeStruct((NPAD, DH), jnp.float32),
            jax.ShapeDtypeStruct((NPAD, DH), jnp.float32),
        ],
    )(u0, o0, v0, cnt, pos_p, wa, wb, wc, b)


def _logits_body(u_ref, o_ref, v_ref, cnt_ref, wf_ref, bf_ref, out_ref):
    cnt = jnp.maximum(cnt_ref[...] - 1.0, 1.0)
    h = jnp.maximum(u_ref[...] + (o_ref[...] - v_ref[...]) / cnt, 0.0)
    out_ref[...] = (jnp.dot(h, wf_ref[...], preferred_element_type=jnp.float32)
                    + bf_ref[...])


def _logits_call(u1, o1, v1, cnt, wf, bf):
    grid = NPAD // MB
    return pl.pallas_call(
        _logits_body,
        grid=(grid,),
        in_specs=[
            pl.BlockSpec((MB, DH), lambda i: (i, 0)),
            pl.BlockSpec((MB, DH), lambda i: (i, 0)),
            pl.BlockSpec((MB, DH), lambda i: (i, 0)),
            pl.BlockSpec((MB, 1), lambda i: (i, 0)),
            pl.BlockSpec((DH, 1), lambda i: (0, 0)),
            pl.BlockSpec((1, 1), lambda i: (0, 0)),
        ],
        out_specs=pl.BlockSpec((MB, 1), lambda i: (i, 0)),
        out_shape=jax.ShapeDtypeStruct((NN, 1), jnp.float32),
    )(u1, o1, v1, cnt, wf, bf)


def _softmax_body(l_ref, p_ref):
    lg = l_ref[...]
    m = jnp.max(lg, axis=0, keepdims=True)
    e = jnp.exp(lg - m)
    p_ref[...] = e / jnp.sum(e, axis=0, keepdims=True)


def _softmax_call(logits):
    return pl.pallas_call(
        _softmax_body,
        out_shape=jax.ShapeDtypeStruct((NN, 1), jnp.float32),
    )(logits)


@jax.jit
def kernel(x, pos, W0, b0, W1, b1, Wf, bf):
    if pos.ndim == 3:
        pos = pos.mean(axis=1)
    x = x.astype(jnp.float32)
    pos = pos.astype(jnp.float32)

    x_p = jnp.pad(x, ((0, NPAD - NN), (0, 0)))
    pos_p = jnp.concatenate(
        [pos, jnp.full((NPAD - NN, 3), 1e9, jnp.float32)], axis=0)
    post = pos.T
    post_p = pos_p.T

    tau = _tau_call(pos_p, post)
    taut = jnp.where(jnp.arange(NPAD)[None, :] < NN,
                     tau.reshape(1, NPAD), jnp.float32(-3e38))

    wa0, wb0, wc0 = W0[:DH], W0[DH:2 * DH], W0[2 * DH:]
    u0, v0 = _pre_call(x_p, pos_p, wa0, wb0, wc0, b0.reshape(1, DH))
    o0, cnt = _aggmm_call(pos_p, post_p, tau, taut, v0)

    wa1, wb1, wc1 = W1[:DH], W1[DH:2 * DH], W1[2 * DH:]
    u1, v1 = _mid_call(u0, o0, v0, cnt, pos_p, wa1, wb1, wc1,
                       b1.reshape(1, DH))
    o1, _ = _aggmm_call(pos_p, post_p, tau, taut, v1)

    logits = _logits_call(u1, o1, v1, cnt, Wf, bf.reshape(1, 1))
    probs = _softmax_call(logits)
    return probs[:, 0]
